# baseline (speedup 1.0000x reference)
import functools

import jax
import jax.numpy as jnp
from jax import lax
from jax.experimental import pallas as pl
from jax.experimental.pallas import tpu as pltpu
from jax.experimental.pallas import tpu_sc as plsc

N = 10000
E = 320000
D = 128
H = 64
NP = 10240
L = 16
NSC = 2
NSUBC = 16
ROWS_PER_TILE = NP // NSUBC

EP = 327680
EPT = EP // NSUBC
SUPER = 1024
NSUPER = EPT // SUPER
CH = 256
NCH = SUPER // CH
SUB = 128
NSUB_CH = CH // SUB
SUBS_PER_SUPER = SUPER // SUB

DCH = 400

_sc_mesh = plsc.VectorSubcoreMesh(core_axis_name="c", subcore_axis_name="s")


def _deg_body(row_hbm, col_hbm, w_hbm, out_hbm, rbuf, cbuf, wbuf, deg_v,
              dsem_a, dsem_b):
    c = lax.axis_index("c")
    s = lax.axis_index("s")
    wid = s * NSC + c
    dsems = (dsem_a, dsem_b)
    zeros = jnp.zeros((L,), jnp.float32)

    def zb(j, _):
        deg_v[pl.ds(j * L, L)] = zeros
        return 0

    lax.fori_loop(0, NP // L, zb, 0)

    epw = E // (NSC * NSUBC)
    nchunk = epw // DCH

    def descs(k, b):
        base = pl.multiple_of(wid * epw + k * DCH, 8)
        return (
            pltpu.make_async_copy(row_hbm.at[pl.ds(base, DCH)],
                                  rbuf.at[b], dsems[b]),
            pltpu.make_async_copy(col_hbm.at[pl.ds(base, DCH)],
                                  cbuf.at[b], dsems[b]),
            pltpu.make_async_copy(w_hbm.at[pl.ds(base, DCH)],
                                  wbuf.at[b], dsems[b]),
        )

    def fire(k, b):
        for dsc in descs(k, b):
            dsc.start()

    def wait(k, b):
        for dsc in descs(k, b):
            dsc.wait()

    def process(b):
        def ib(i, _):
            r = rbuf[b, pl.ds(i * L, L)]
            cv = cbuf[b, pl.ds(i * L, L)]
            wv = wbuf[b, pl.ds(i * L, L)]
            wsel = jnp.where(r == cv, 0.0, wv)
            plsc.addupdate_scatter(deg_v, [r], wsel)
            return 0

        lax.fori_loop(0, DCH // L, ib, 0)

    fire(0, 0)

    def pair(g, _):
        for b in (0, 1):
            k = 2 * g + b
            fire(k + 1, 1 - b)
            wait(k, b)
            process(b)
        return 0

    lax.fori_loop(0, (nchunk - 1) // 2, pair, 0)
    wait(nchunk - 1, 0)
    process(0)
    pltpu.sync_copy(deg_v, out_hbm.at[wid])


_sc_params = pltpu.CompilerParams(needs_layout_passes=False,
                                  use_tc_tiling_on_sc=False)

_deg_call = pl.kernel(
    _deg_body,
    out_type=jax.ShapeDtypeStruct((NSC * NSUBC, NP), jnp.float32),
    mesh=_sc_mesh,
    compiler_params=_sc_params,
    scratch_types=[
        pltpu.VMEM((2, DCH), jnp.int32),
        pltpu.VMEM((2, DCH), jnp.int32),
        pltpu.VMEM((2, DCH), jnp.float32),
        pltpu.VMEM((NP,), jnp.float32),
        pltpu.SemaphoreType.DMA,
        pltpu.SemaphoreType.DMA,
    ],
)


def _dis_body(degp_ref, dis_ref):
    deg = jnp.sum(degp_ref[...], axis=0, keepdims=True)
    dis_ref[...] = jnp.where(
        deg > 0, lax.rsqrt(jnp.maximum(deg, 1e-12)), 0.0)


def _dis_call(degp):
    return pl.pallas_call(
        _dis_body,
        out_shape=jax.ShapeDtypeStruct((1, NP), jnp.float32),
    )(degp)


def _lmv_body(src_hbm, row_hbm, col2d_hbm, w_hbm, dis_hbm, out_hbm,
              dis_v, ridx_s, w_s, cidx_s, gidx_s, wn_s, rows_a, rows_b,
              acc, lsem_a, lsem_b, gsem_a, gsem_b, ssem_a, ssem_b):
    c = lax.axis_index("c")
    s = lax.axis_index("s")
    lsems = (lsem_a, lsem_b)
    gsems = (gsem_a, gsem_b)
    ssems = (ssem_a, ssem_b)
    rows_bufs = (rows_a, rows_b)
    pltpu.sync_copy(dis_hbm, dis_v)
    zeros = jnp.zeros((L,), jnp.float32)

    def zb(i, _):
        for j in range(H // L):
            rows_a[i, pl.ds(j * L, L)] = zeros
        return 0

    lax.fori_loop(0, CH, zb, 0)
    off = 0
    while off < ROWS_PER_TILE:
        blk = min(CH, ROWS_PER_TILE - off)
        pltpu.sync_copy(rows_a.at[pl.ds(0, blk)],
                        acc.at[pl.ds(s * ROWS_PER_TILE + off, blk)])
        off += blk
    plsc.subcore_barrier()

    ebase = s * EPT
    coff = c * NP
    zero_i = jnp.zeros((L,), jnp.int32)

    def linear_descs(p, b):
        base = pl.multiple_of(ebase + p * SUPER, 8)
        rbase = pl.multiple_of((ebase + p * SUPER) // SUB, 8)
        return (
            pltpu.make_async_copy(row_hbm.at[pl.ds(base, SUPER)],
                                  ridx_s.at[b], lsems[b]),
            pltpu.make_async_copy(w_hbm.at[pl.ds(base, SUPER)],
                                  w_s.at[b], lsems[b]),
            pltpu.make_async_copy(col2d_hbm.at[pl.ds(rbase, SUBS_PER_SUPER)],
                                  cidx_s.at[b], lsems[b]),
        )

    def fire_linear(p, b):
        for dsc in linear_descs(p, b):
            dsc.start()

    def wait_linear(p, b):
        for dsc in linear_descs(p, b):
            dsc.wait()

    def gidx_compute(b):
        def gb(i):
            jb = i // (SUB // L)
            i2 = lax.rem(i, SUB // L)
            r = ridx_s[b, pl.ds(i * L, L)]
            gidx_s[b, jb, pl.ds(i2 * L, L)] = r + coff

        plsc.parallel_loop(0, SUPER // L, 1, unroll=4)(gb)

    def wn_compute(b):
        def wb(i):
            jb = i // (SUB // L)
            i2 = lax.rem(i, SUB // L)
            r = ridx_s[b, pl.ds(i * L, L)]
            cv = cidx_s[b, jb, pl.ds(i2 * L, L)]
            wv = w_s[b, pl.ds(i * L, L)]
            dr = plsc.load_gather(dis_v, [r])
            dc = plsc.load_gather(dis_v, [cv])
            wn = jnp.where(r == cv, 0.0, -dr * wv * dc)
            wn_s[b, pl.ds(i * L, L)] = wn

        plsc.parallel_loop(0, SUPER // L, 1, unroll=4)(wb)

    def gather_descs(b, k, rbuf, gs):
        return [
            pltpu.make_async_copy(src_hbm.at[gidx_s.at[b, k * NSUB_CH + j]],
                                  rbuf.at[pl.ds(j * SUB, SUB)], gs)
            for j in range(NSUB_CH)
        ]

    def scale_chunk(b, k, rbuf):
        koff = k * CH

        def sb(i):
            wni = plsc.load_gather(wn_s.at[b], [zero_i + (koff + i)])
            for jj in range(H // L):
                rbuf[i, pl.ds(jj * L, L)] = rbuf[i, pl.ds(jj * L, L)] * wni

        plsc.parallel_loop(0, CH, 1, unroll=8)(sb)

    def scatter_descs(b, k, rbuf):
        return [
            pltpu.async_copy(rbuf.at[pl.ds(j * SUB, SUB)],
                             acc.at[cidx_s.at[b, k * NSUB_CH + j]],
                             ssems[k % 2], add=True)
            for j in range(NSUB_CH)
        ]

    def do_super(p, b):
        @pl.when(p + 1 < NSUPER)
        def _():
            fire_linear(p + 1, 1 - b)

        wait_linear(p, b)
        gidx_compute(b)
        for dsc in gather_descs(b, 0, rows_a, gsems[0]):
            dsc.start()
        wn_compute(b)
        pending = {}
        for k in range(NCH):
            rbuf = rows_bufs[k % 2]
            gs = gsems[k % 2]
            if k + 1 < NCH:
                if k - 1 >= 0:
                    for dsc in pending.pop(k - 1):
                        dsc.wait()
                for dsc in gather_descs(b, k + 1, rows_bufs[(k + 1) % 2],
                                        gsems[(k + 1) % 2]):
                    dsc.start()
            for dsc in gather_descs(b, k, rbuf, gs):
                dsc.wait()
            scale_chunk(b, k, rbuf)
            pending[k] = scatter_descs(b, k, rbuf)
        for k in sorted(pending):
            for dsc in pending.pop(k):
                dsc.wait()

    fire_linear(0, 0)

    def gloop(g, _):
        do_super(2 * g, 0)
        do_super(2 * g + 1, 1)
        return 0

    lax.fori_loop(0, NSUPER // 2, gloop, 0)
    plsc.subcore_barrier()
    pltpu.sync_copy(acc.at[pl.ds(s * ROWS_PER_TILE, ROWS_PER_TILE)],
                    out_hbm.at[pl.ds(coff + s * ROWS_PER_TILE,
                                     ROWS_PER_TILE)])


_lmv_call = pl.kernel(
    _lmv_body,
    out_type=jax.ShapeDtypeStruct((NSC * NP, H), jnp.float32),
    mesh=_sc_mesh,
    compiler_params=_sc_params,
    scratch_types=[
        pltpu.VMEM((NP,), jnp.float32),
        pltpu.VMEM((2, SUPER), jnp.int32),
        pltpu.VMEM((2, SUPER), jnp.float32),
        pltpu.VMEM((2, SUBS_PER_SUPER, SUB), jnp.int32),
        pltpu.VMEM((2, SUBS_PER_SUPER, SUB), jnp.int32),
        pltpu.VMEM((2, SUPER), jnp.float32),
        pltpu.VMEM((CH, H), jnp.float32),
        pltpu.VMEM((CH, H), jnp.float32),
        pltpu.VMEM_SHARED((NP, H), jnp.float32),
        pltpu.SemaphoreType.DMA,
        pltpu.SemaphoreType.DMA,
        pltpu.SemaphoreType.DMA,
        pltpu.SemaphoreType.DMA,
        pltpu.SemaphoreType.DMA,
        pltpu.SemaphoreType.DMA,
    ],
)


RB = 1000


def _dense_body(x_ref, t1_ref, t2_ref, w0_ref, w1_ref, w2_ref,
                b_ref, g_ref, be_ref, o_ref):
    xb = x_ref[...]
    t1 = jnp.concatenate([t1_ref[0], t1_ref[1]], axis=1)
    t2s = jnp.concatenate([t2_ref[0], t2_ref[1]], axis=1)
    t2 = 2.0 * t2s - xb
    acc = (jnp.dot(xb, w0_ref[...], preferred_element_type=jnp.float32)
           + jnp.dot(t1, w1_ref[...], preferred_element_type=jnp.float32)
           + jnp.dot(t2, w2_ref[...], preferred_element_type=jnp.float32)
           + b_ref[...])
    acc = jnp.maximum(acc, 0.0)
    mu = jnp.mean(acc, axis=1, keepdims=True)
    var = jnp.mean((acc - mu) ** 2, axis=1, keepdims=True)
    o_ref[...] = ((acc - mu) * lax.rsqrt(var + 1e-5) * g_ref[...]
                  + be_ref[...])


def _dense_call(x, t1, t2, W0, W1, W2, b2, g2, be2):
    grid = (N // RB,)
    return pl.pallas_call(
        _dense_body,
        grid=grid,
        in_specs=[
            pl.BlockSpec((RB, D), lambda i: (i, 0)),
            pl.BlockSpec((2, RB, H), lambda i: (0, i, 0)),
            pl.BlockSpec((2, RB, H), lambda i: (0, i, 0)),
            pl.BlockSpec((D, D), lambda i: (0, 0)),
            pl.BlockSpec((D, D), lambda i: (0, 0)),
            pl.BlockSpec((D, D), lambda i: (0, 0)),
            pl.BlockSpec((1, D), lambda i: (0, 0)),
            pl.BlockSpec((1, D), lambda i: (0, 0)),
            pl.BlockSpec((1, D), lambda i: (0, 0)),
        ],
        out_specs=pl.BlockSpec((RB, D), lambda i: (i, 0)),
        out_shape=jax.ShapeDtypeStruct((N, D), jnp.float32),
    )(x, t1, t2, W0, W1, W2, b2, g2, be2)


def kernel(x, edge_index, edge_weight, W0, W1, W2, b, gamma, beta):
    row = edge_index[0]
    col = edge_index[1]
    w = edge_weight

    degp = _deg_call(row, col, w)
    dis = _dis_call(degp).reshape(NP)

    xflat = jnp.zeros((NSC * NP, H), jnp.float32)
    xflat = lax.dynamic_update_slice(xflat, x[:, :H], (0, 0))
    xflat = lax.dynamic_update_slice(xflat, x[:, H:], (NP, 0))

    pad = EP - E
    rowp = jnp.concatenate([row, jnp.zeros((pad,), jnp.int32)])
    colp = jnp.concatenate([col, jnp.zeros((pad,), jnp.int32)])
    wp = jnp.concatenate([w, jnp.zeros((pad,), jnp.float32)])
    col2d = colp.reshape(EP // SUB, SUB)

    tx1_flat = _lmv_call(xflat, rowp, col2d, wp, dis)
    tx2_flat = _lmv_call(tx1_flat, rowp, col2d, wp, dis)

    t1 = tx1_flat.reshape(NSC, NP, H)[:, :N]
    t2 = tx2_flat.reshape(NSC, NP, H)[:, :N]

    out = _dense_call(x, t1, t2, W0, W1, W2,
                      b.reshape(1, D), gamma.reshape(1, D),
                      beta.reshape(1, D))
    return out


# speedup vs baseline: 1.0422x; 1.0422x over previous
import functools

import jax
import jax.numpy as jnp
from jax import lax
from jax.experimental import pallas as pl
from jax.experimental.pallas import tpu as pltpu
from jax.experimental.pallas import tpu_sc as plsc

N = 10000
E = 320000
D = 128
H = 64
NP = 10240
L = 16
NSC = 2
NSUBC = 16
ROWS_PER_TILE = NP // NSUBC

EP = 327680
EPT = EP // NSUBC
SUPER = 1024
NSUPER = EPT // SUPER
CH = 128
NCH = SUPER // CH
SUB = 128
SUBS_PER_SUPER = SUPER // SUB
NRING = 4

DCH = 400

_sc_mesh = plsc.VectorSubcoreMesh(core_axis_name="c", subcore_axis_name="s")


def _deg_body(row_hbm, col_hbm, w_hbm, out_hbm, rbuf, cbuf, wbuf, deg_v,
              dsem_a, dsem_b):
    c = lax.axis_index("c")
    s = lax.axis_index("s")
    wid = s * NSC + c
    dsems = (dsem_a, dsem_b)
    zeros = jnp.zeros((L,), jnp.float32)

    def zb(j, _):
        deg_v[pl.ds(j * L, L)] = zeros
        return 0

    lax.fori_loop(0, NP // L, zb, 0)

    epw = E // (NSC * NSUBC)
    nchunk = epw // DCH

    def descs(k, b):
        base = pl.multiple_of(wid * epw + k * DCH, 8)
        return (
            pltpu.make_async_copy(row_hbm.at[pl.ds(base, DCH)],
                                  rbuf.at[b], dsems[b]),
            pltpu.make_async_copy(col_hbm.at[pl.ds(base, DCH)],
                                  cbuf.at[b], dsems[b]),
            pltpu.make_async_copy(w_hbm.at[pl.ds(base, DCH)],
                                  wbuf.at[b], dsems[b]),
        )

    def fire(k, b):
        for dsc in descs(k, b):
            dsc.start()

    def wait(k, b):
        for dsc in descs(k, b):
            dsc.wait()

    def process(b):
        def ib(i, _):
            r = rbuf[b, pl.ds(i * L, L)]
            cv = cbuf[b, pl.ds(i * L, L)]
            wv = wbuf[b, pl.ds(i * L, L)]
            wsel = jnp.where(r == cv, 0.0, wv)
            plsc.addupdate_scatter(deg_v, [r], wsel)
            return 0

        lax.fori_loop(0, DCH // L, ib, 0)

    fire(0, 0)

    def pair(g, _):
        for b in (0, 1):
            k = 2 * g + b
            fire(k + 1, 1 - b)
            wait(k, b)
            process(b)
        return 0

    lax.fori_loop(0, (nchunk - 1) // 2, pair, 0)
    wait(nchunk - 1, 0)
    process(0)
    pltpu.sync_copy(deg_v, out_hbm.at[wid])


_sc_params = pltpu.CompilerParams(needs_layout_passes=False,
                                  use_tc_tiling_on_sc=False)

_deg_call = pl.kernel(
    _deg_body,
    out_type=jax.ShapeDtypeStruct((NSC * NSUBC, NP), jnp.float32),
    mesh=_sc_mesh,
    compiler_params=_sc_params,
    scratch_types=[
        pltpu.VMEM((2, DCH), jnp.int32),
        pltpu.VMEM((2, DCH), jnp.int32),
        pltpu.VMEM((2, DCH), jnp.float32),
        pltpu.VMEM((NP,), jnp.float32),
        pltpu.SemaphoreType.DMA,
        pltpu.SemaphoreType.DMA,
    ],
)


def _dis_body(degp_ref, dis_ref):
    deg = jnp.sum(degp_ref[...], axis=0, keepdims=True)
    dis_ref[...] = jnp.where(
        deg > 0, lax.rsqrt(jnp.maximum(deg, 1e-12)), 0.0)


def _dis_call(degp):
    return pl.pallas_call(
        _dis_body,
        out_shape=jax.ShapeDtypeStruct((1, NP), jnp.float32),
    )(degp)


def _lmv_body(src_hbm, row_hbm, col2d_hbm, w_hbm, dis_hbm, out_hbm,
              dis_v, ridx_s, w_s, cidx_s, gidx_s, wn_s,
              rows_a, rows_b, rows_c, rows_d,
              acc, lsem_a, lsem_b, gsem_a, gsem_b, gsem_c, gsem_d,
              ssem_a, ssem_b, ssem_c, ssem_d):
    c = lax.axis_index("c")
    s = lax.axis_index("s")
    lsems = (lsem_a, lsem_b)
    gsems = (gsem_a, gsem_b, gsem_c, gsem_d)
    ssems = (ssem_a, ssem_b, ssem_c, ssem_d)
    rows_bufs = (rows_a, rows_b, rows_c, rows_d)
    pltpu.sync_copy(dis_hbm, dis_v)
    zeros = jnp.zeros((L,), jnp.float32)

    def zb(i, _):
        for j in range(H // L):
            rows_a[i, pl.ds(j * L, L)] = zeros
        return 0

    lax.fori_loop(0, CH, zb, 0)
    off = 0
    while off < ROWS_PER_TILE:
        blk = min(CH, ROWS_PER_TILE - off)
        pltpu.sync_copy(rows_a.at[pl.ds(0, blk)],
                        acc.at[pl.ds(s * ROWS_PER_TILE + off, blk)])
        off += blk
    plsc.subcore_barrier()

    ebase = s * EPT
    coff = c * NP
    zero_i = jnp.zeros((L,), jnp.int32)

    def linear_descs(p, b):
        base = pl.multiple_of(ebase + p * SUPER, 8)
        rbase = pl.multiple_of((ebase + p * SUPER) // SUB, 8)
        return (
            pltpu.make_async_copy(row_hbm.at[pl.ds(base, SUPER)],
                                  ridx_s.at[b], lsems[b]),
            pltpu.make_async_copy(w_hbm.at[pl.ds(base, SUPER)],
                                  w_s.at[b], lsems[b]),
            pltpu.make_async_copy(col2d_hbm.at[pl.ds(rbase, SUBS_PER_SUPER)],
                                  cidx_s.at[b], lsems[b]),
        )

    def fire_linear(p, b):
        for dsc in linear_descs(p, b):
            dsc.start()

    def wait_linear(p, b):
        for dsc in linear_descs(p, b):
            dsc.wait()

    def gidx_compute(b):
        def gb(i):
            jb = i // (SUB // L)
            i2 = lax.rem(i, SUB // L)
            r = ridx_s[b, pl.ds(i * L, L)]
            gidx_s[b, jb, pl.ds(i2 * L, L)] = r + coff

        plsc.parallel_loop(0, SUPER // L, 1, unroll=4)(gb)

    def wn_compute(b):
        def wb(i):
            jb = i // (SUB // L)
            i2 = lax.rem(i, SUB // L)
            r = ridx_s[b, pl.ds(i * L, L)]
            cv = cidx_s[b, jb, pl.ds(i2 * L, L)]
            wv = w_s[b, pl.ds(i * L, L)]
            dr = plsc.load_gather(dis_v, [r])
            dc = plsc.load_gather(dis_v, [cv])
            wn = jnp.where(r == cv, 0.0, -dr * wv * dc)
            wn_s[b, pl.ds(i * L, L)] = wn

        plsc.parallel_loop(0, SUPER // L, 1, unroll=4)(wb)

    def gather_desc(b, k):
        return pltpu.make_async_copy(src_hbm.at[gidx_s.at[b, k]],
                                     rows_bufs[k % NRING], gsems[k % NRING])

    def scale_chunk(b, k, rbuf):
        koff = k * CH

        def sb(i):
            wni = plsc.load_gather(wn_s.at[b], [zero_i + (koff + i)])
            for jj in range(H // L):
                rbuf[i, pl.ds(jj * L, L)] = rbuf[i, pl.ds(jj * L, L)] * wni

        plsc.parallel_loop(0, CH, 1, unroll=8)(sb)

    def scatter_desc(b, k):
        return pltpu.async_copy(rows_bufs[k % NRING],
                                acc.at[cidx_s.at[b, k]],
                                ssems[k % NRING], add=True)

    def scatter_wait(b, k):
        pltpu.make_async_copy(rows_bufs[k % NRING],
                              acc.at[cidx_s.at[b, k]],
                              ssems[k % NRING]).wait()

    def do_super(p, b):
        @pl.when(p + 1 < NSUPER)
        def _():
            fire_linear(p + 1, 1 - b)

        wait_linear(p, b)
        gidx_compute(b)
        gather_desc(b, 0).start()
        gather_desc(b, 1).start()
        wn_compute(b)
        for k in range(NCH):
            if k + 2 < NCH:
                if k - 2 >= 0:
                    scatter_wait(b, k - 2)
                gather_desc(b, k + 2).start()
            gather_desc(b, k).wait()
            scale_chunk(b, k, rows_bufs[k % NRING])
            scatter_desc(b, k)
        for k in range(NCH - NRING, NCH):
            scatter_wait(b, k)

    fire_linear(0, 0)

    def gloop(g, _):
        do_super(2 * g, 0)
        do_super(2 * g + 1, 1)
        return 0

    lax.fori_loop(0, NSUPER // 2, gloop, 0)
    plsc.subcore_barrier()
    pltpu.sync_copy(acc.at[pl.ds(s * ROWS_PER_TILE, ROWS_PER_TILE)],
                    out_hbm.at[pl.ds(coff + s * ROWS_PER_TILE,
                                     ROWS_PER_TILE)])


_lmv_call = pl.kernel(
    _lmv_body,
    out_type=jax.ShapeDtypeStruct((NSC * NP, H), jnp.float32),
    mesh=_sc_mesh,
    compiler_params=_sc_params,
    scratch_types=[
        pltpu.VMEM((NP,), jnp.float32),
        pltpu.VMEM((2, SUPER), jnp.int32),
        pltpu.VMEM((2, SUPER), jnp.float32),
        pltpu.VMEM((2, SUBS_PER_SUPER, SUB), jnp.int32),
        pltpu.VMEM((2, SUBS_PER_SUPER, SUB), jnp.int32),
        pltpu.VMEM((2, SUPER), jnp.float32),
        pltpu.VMEM((CH, H), jnp.float32),
        pltpu.VMEM((CH, H), jnp.float32),
        pltpu.VMEM((CH, H), jnp.float32),
        pltpu.VMEM((CH, H), jnp.float32),
        pltpu.VMEM_SHARED((NP, H), jnp.float32),
    ] + [pltpu.SemaphoreType.DMA] * 10,
)


RB = 1000


def _dense_body(x_ref, t1_ref, t2_ref, w0_ref, w1_ref, w2_ref,
                b_ref, g_ref, be_ref, o_ref):
    xb = x_ref[...]
    t1 = jnp.concatenate([t1_ref[0], t1_ref[1]], axis=1)
    t2s = jnp.concatenate([t2_ref[0], t2_ref[1]], axis=1)
    t2 = 2.0 * t2s - xb
    acc = (jnp.dot(xb, w0_ref[...], preferred_element_type=jnp.float32)
           + jnp.dot(t1, w1_ref[...], preferred_element_type=jnp.float32)
           + jnp.dot(t2, w2_ref[...], preferred_element_type=jnp.float32)
           + b_ref[...])
    acc = jnp.maximum(acc, 0.0)
    mu = jnp.mean(acc, axis=1, keepdims=True)
    var = jnp.mean((acc - mu) ** 2, axis=1, keepdims=True)
    o_ref[...] = ((acc - mu) * lax.rsqrt(var + 1e-5) * g_ref[...]
                  + be_ref[...])


def _dense_call(x, t1, t2, W0, W1, W2, b2, g2, be2):
    grid = (N // RB,)
    return pl.pallas_call(
        _dense_body,
        grid=grid,
        in_specs=[
            pl.BlockSpec((RB, D), lambda i: (i, 0)),
            pl.BlockSpec((2, RB, H), lambda i: (0, i, 0)),
            pl.BlockSpec((2, RB, H), lambda i: (0, i, 0)),
            pl.BlockSpec((D, D), lambda i: (0, 0)),
            pl.BlockSpec((D, D), lambda i: (0, 0)),
            pl.BlockSpec((D, D), lambda i: (0, 0)),
            pl.BlockSpec((1, D), lambda i: (0, 0)),
            pl.BlockSpec((1, D), lambda i: (0, 0)),
            pl.BlockSpec((1, D), lambda i: (0, 0)),
        ],
        out_specs=pl.BlockSpec((RB, D), lambda i: (i, 0)),
        out_shape=jax.ShapeDtypeStruct((N, D), jnp.float32),
    )(x, t1, t2, W0, W1, W2, b2, g2, be2)


def kernel(x, edge_index, edge_weight, W0, W1, W2, b, gamma, beta):
    row = edge_index[0]
    col = edge_index[1]
    w = edge_weight

    degp = _deg_call(row, col, w)
    dis = _dis_call(degp).reshape(NP)

    xflat = jnp.zeros((NSC * NP, H), jnp.float32)
    xflat = lax.dynamic_update_slice(xflat, x[:, :H], (0, 0))
    xflat = lax.dynamic_update_slice(xflat, x[:, H:], (NP, 0))

    pad = EP - E
    rowp = jnp.concatenate([row, jnp.zeros((pad,), jnp.int32)])
    colp = jnp.concatenate([col, jnp.zeros((pad,), jnp.int32)])
    wp = jnp.concatenate([w, jnp.zeros((pad,), jnp.float32)])
    col2d = colp.reshape(EP // SUB, SUB)

    tx1_flat = _lmv_call(xflat, rowp, col2d, wp, dis)
    tx2_flat = _lmv_call(tx1_flat, rowp, col2d, wp, dis)

    t1 = tx1_flat.reshape(NSC, NP, H)[:, :N]
    t2 = tx2_flat.reshape(NSC, NP, H)[:, :N]

    out = _dense_call(x, t1, t2, W0, W1, W2,
                      b.reshape(1, D), gamma.reshape(1, D),
                      beta.reshape(1, D))
    return out


# speedup vs baseline: 1.0684x; 1.0251x over previous
import functools

import jax
import jax.numpy as jnp
from jax import lax
from jax.experimental import pallas as pl
from jax.experimental.pallas import tpu as pltpu
from jax.experimental.pallas import tpu_sc as plsc

N = 10000
E = 320000
D = 128
H = 64
NP = 10240
L = 16
NSC = 2
NSUBC = 16
ROWS_PER_TILE = NP // NSUBC

EP = 327680
EPT = EP // NSUBC
SUPER = 2048
NSUPER = EPT // SUPER
CH = 128
NCH = SUPER // CH
SUB = 128
SUBS_PER_SUPER = SUPER // SUB
NRING = 4

DCH = 400

_sc_mesh = plsc.VectorSubcoreMesh(core_axis_name="c", subcore_axis_name="s")


def _deg_body(row_hbm, col_hbm, w_hbm, out_hbm, rbuf, cbuf, wbuf, deg_v,
              dsem_a, dsem_b):
    c = lax.axis_index("c")
    s = lax.axis_index("s")
    wid = s * NSC + c
    dsems = (dsem_a, dsem_b)
    zeros = jnp.zeros((L,), jnp.float32)

    def zb(j, _):
        deg_v[pl.ds(j * L, L)] = zeros
        return 0

    lax.fori_loop(0, NP // L, zb, 0)

    epw = E // (NSC * NSUBC)
    nchunk = epw // DCH

    def descs(k, b):
        base = pl.multiple_of(wid * epw + k * DCH, 8)
        return (
            pltpu.make_async_copy(row_hbm.at[pl.ds(base, DCH)],
                                  rbuf.at[b], dsems[b]),
            pltpu.make_async_copy(col_hbm.at[pl.ds(base, DCH)],
                                  cbuf.at[b], dsems[b]),
            pltpu.make_async_copy(w_hbm.at[pl.ds(base, DCH)],
                                  wbuf.at[b], dsems[b]),
        )

    def fire(k, b):
        for dsc in descs(k, b):
            dsc.start()

    def wait(k, b):
        for dsc in descs(k, b):
            dsc.wait()

    def process(b):
        def ib(i, _):
            r = rbuf[b, pl.ds(i * L, L)]
            cv = cbuf[b, pl.ds(i * L, L)]
            wv = wbuf[b, pl.ds(i * L, L)]
            wsel = jnp.where(r == cv, 0.0, wv)
            plsc.addupdate_scatter(deg_v, [r], wsel)
            return 0

        lax.fori_loop(0, DCH // L, ib, 0)

    fire(0, 0)

    def pair(g, _):
        for b in (0, 1):
            k = 2 * g + b
            fire(k + 1, 1 - b)
            wait(k, b)
            process(b)
        return 0

    lax.fori_loop(0, (nchunk - 1) // 2, pair, 0)
    wait(nchunk - 1, 0)
    process(0)
    pltpu.sync_copy(deg_v, out_hbm.at[wid])


_sc_params = pltpu.CompilerParams(needs_layout_passes=False,
                                  use_tc_tiling_on_sc=False)

_deg_call = pl.kernel(
    _deg_body,
    out_type=jax.ShapeDtypeStruct((NSC * NSUBC, NP), jnp.float32),
    mesh=_sc_mesh,
    compiler_params=_sc_params,
    scratch_types=[
        pltpu.VMEM((2, DCH), jnp.int32),
        pltpu.VMEM((2, DCH), jnp.int32),
        pltpu.VMEM((2, DCH), jnp.float32),
        pltpu.VMEM((NP,), jnp.float32),
        pltpu.SemaphoreType.DMA,
        pltpu.SemaphoreType.DMA,
    ],
)


def _dis_body(degp_ref, dis_ref):
    deg = jnp.sum(degp_ref[...], axis=0, keepdims=True)
    dis_ref[...] = jnp.where(
        deg > 0, lax.rsqrt(jnp.maximum(deg, 1e-12)), 0.0)


def _dis_call(degp):
    return pl.pallas_call(
        _dis_body,
        out_shape=jax.ShapeDtypeStruct((1, NP), jnp.float32),
    )(degp)


def _lmv_body(src_hbm, row_hbm, col2d_hbm, w_hbm, dis_hbm, out_hbm,
              dis_v, ridx_s, w_s, cidx_s, gidx_s, wn_s,
              rows_a, rows_b, rows_c, rows_d,
              acc, lsem_a, lsem_b, gsem_a, gsem_b, gsem_c, gsem_d,
              ssem_a, ssem_b, ssem_c, ssem_d):
    c = lax.axis_index("c")
    s = lax.axis_index("s")
    lsems = (lsem_a, lsem_b)
    gsems = (gsem_a, gsem_b, gsem_c, gsem_d)
    ssems = (ssem_a, ssem_b, ssem_c, ssem_d)
    rows_bufs = (rows_a, rows_b, rows_c, rows_d)
    pltpu.sync_copy(dis_hbm, dis_v)
    zeros = jnp.zeros((L,), jnp.float32)

    def zb(i, _):
        for j in range(H // L):
            rows_a[i, pl.ds(j * L, L)] = zeros
        return 0

    lax.fori_loop(0, CH, zb, 0)
    off = 0
    while off < ROWS_PER_TILE:
        blk = min(CH, ROWS_PER_TILE - off)
        pltpu.sync_copy(rows_a.at[pl.ds(0, blk)],
                        acc.at[pl.ds(s * ROWS_PER_TILE + off, blk)])
        off += blk
    plsc.subcore_barrier()

    ebase = s * EPT
    coff = c * NP
    zero_i = jnp.zeros((L,), jnp.int32)

    def linear_descs(p, b):
        base = pl.multiple_of(ebase + p * SUPER, 8)
        rbase = pl.multiple_of((ebase + p * SUPER) // SUB, 8)
        return (
            pltpu.make_async_copy(row_hbm.at[pl.ds(base, SUPER)],
                                  ridx_s.at[b], lsems[b]),
            pltpu.make_async_copy(w_hbm.at[pl.ds(base, SUPER)],
                                  w_s.at[b], lsems[b]),
            pltpu.make_async_copy(col2d_hbm.at[pl.ds(rbase, SUBS_PER_SUPER)],
                                  cidx_s.at[b], lsems[b]),
        )

    def fire_linear(p, b):
        for dsc in linear_descs(p, b):
            dsc.start()

    def wait_linear(p, b):
        for dsc in linear_descs(p, b):
            dsc.wait()

    def gidx_compute(b):
        def gb(i):
            jb = i // (SUB // L)
            i2 = lax.rem(i, SUB // L)
            r = ridx_s[b, pl.ds(i * L, L)]
            gidx_s[b, jb, pl.ds(i2 * L, L)] = r + coff

        plsc.parallel_loop(0, SUPER // L, 1, unroll=4)(gb)

    def wn_compute(b):
        def wb(i):
            jb = i // (SUB // L)
            i2 = lax.rem(i, SUB // L)
            r = ridx_s[b, pl.ds(i * L, L)]
            cv = cidx_s[b, jb, pl.ds(i2 * L, L)]
            wv = w_s[b, pl.ds(i * L, L)]
            dr = plsc.load_gather(dis_v, [r])
            dc = plsc.load_gather(dis_v, [cv])
            wn = jnp.where(r == cv, 0.0, -dr * wv * dc)
            wn_s[b, pl.ds(i * L, L)] = wn

        plsc.parallel_loop(0, SUPER // L, 1, unroll=4)(wb)

    def gather_desc(b, k):
        return pltpu.make_async_copy(src_hbm.at[gidx_s.at[b, k]],
                                     rows_bufs[k % NRING], gsems[k % NRING])

    def scale_chunk(b, k, rbuf):
        koff = k * CH

        def sb(i):
            wni = plsc.load_gather(wn_s.at[b], [zero_i + (koff + i)])
            for jj in range(H // L):
                rbuf[i, pl.ds(jj * L, L)] = rbuf[i, pl.ds(jj * L, L)] * wni

        plsc.parallel_loop(0, CH, 1, unroll=8)(sb)

    def scatter_desc(b, k):
        return pltpu.async_copy(rows_bufs[k % NRING],
                                acc.at[cidx_s.at[b, k]],
                                ssems[k % NRING], add=True)

    def scatter_wait(b, k):
        pltpu.make_async_copy(rows_bufs[k % NRING],
                              acc.at[cidx_s.at[b, k]],
                              ssems[k % NRING]).wait()

    def do_super(p, b):
        @pl.when(p + 1 < NSUPER)
        def _():
            fire_linear(p + 1, 1 - b)

        wait_linear(p, b)
        gidx_compute(b)
        gather_desc(b, 0).start()
        gather_desc(b, 1).start()
        wn_compute(b)
        for k in range(NCH):
            if k + 2 < NCH:
                if k - 2 >= 0:
                    scatter_wait(b, k - 2)
                gather_desc(b, k + 2).start()
            gather_desc(b, k).wait()
            scale_chunk(b, k, rows_bufs[k % NRING])
            scatter_desc(b, k)
        for k in range(NCH - NRING, NCH):
            scatter_wait(b, k)

    fire_linear(0, 0)

    def gloop(g, _):
        do_super(2 * g, 0)
        do_super(2 * g + 1, 1)
        return 0

    lax.fori_loop(0, NSUPER // 2, gloop, 0)
    plsc.subcore_barrier()
    pltpu.sync_copy(acc.at[pl.ds(s * ROWS_PER_TILE, ROWS_PER_TILE)],
                    out_hbm.at[pl.ds(coff + s * ROWS_PER_TILE,
                                     ROWS_PER_TILE)])


_lmv_call = pl.kernel(
    _lmv_body,
    out_type=jax.ShapeDtypeStruct((NSC * NP, H), jnp.float32),
    mesh=_sc_mesh,
    compiler_params=_sc_params,
    scratch_types=[
        pltpu.VMEM((NP,), jnp.float32),
        pltpu.VMEM((2, SUPER), jnp.int32),
        pltpu.VMEM((2, SUPER), jnp.float32),
        pltpu.VMEM((2, SUBS_PER_SUPER, SUB), jnp.int32),
        pltpu.VMEM((2, SUBS_PER_SUPER, SUB), jnp.int32),
        pltpu.VMEM((2, SUPER), jnp.float32),
        pltpu.VMEM((CH, H), jnp.float32),
        pltpu.VMEM((CH, H), jnp.float32),
        pltpu.VMEM((CH, H), jnp.float32),
        pltpu.VMEM((CH, H), jnp.float32),
        pltpu.VMEM_SHARED((NP, H), jnp.float32),
    ] + [pltpu.SemaphoreType.DMA] * 10,
)


RB = 1000


def _dense_body(x_ref, t1_ref, t2_ref, w0_ref, w1_ref, w2_ref,
                b_ref, g_ref, be_ref, o_ref):
    xb = x_ref[...]
    t1 = jnp.concatenate([t1_ref[0], t1_ref[1]], axis=1)
    t2s = jnp.concatenate([t2_ref[0], t2_ref[1]], axis=1)
    t2 = 2.0 * t2s - xb
    acc = (jnp.dot(xb, w0_ref[...], preferred_element_type=jnp.float32)
           + jnp.dot(t1, w1_ref[...], preferred_element_type=jnp.float32)
           + jnp.dot(t2, w2_ref[...], preferred_element_type=jnp.float32)
           + b_ref[...])
    acc = jnp.maximum(acc, 0.0)
    mu = jnp.mean(acc, axis=1, keepdims=True)
    var = jnp.mean((acc - mu) ** 2, axis=1, keepdims=True)
    o_ref[...] = ((acc - mu) * lax.rsqrt(var + 1e-5) * g_ref[...]
                  + be_ref[...])


def _dense_call(x, t1, t2, W0, W1, W2, b2, g2, be2):
    grid = (N // RB,)
    return pl.pallas_call(
        _dense_body,
        grid=grid,
        in_specs=[
            pl.BlockSpec((RB, D), lambda i: (i, 0)),
            pl.BlockSpec((2, RB, H), lambda i: (0, i, 0)),
            pl.BlockSpec((2, RB, H), lambda i: (0, i, 0)),
            pl.BlockSpec((D, D), lambda i: (0, 0)),
            pl.BlockSpec((D, D), lambda i: (0, 0)),
            pl.BlockSpec((D, D), lambda i: (0, 0)),
            pl.BlockSpec((1, D), lambda i: (0, 0)),
            pl.BlockSpec((1, D), lambda i: (0, 0)),
            pl.BlockSpec((1, D), lambda i: (0, 0)),
        ],
        out_specs=pl.BlockSpec((RB, D), lambda i: (i, 0)),
        out_shape=jax.ShapeDtypeStruct((N, D), jnp.float32),
    )(x, t1, t2, W0, W1, W2, b2, g2, be2)


def kernel(x, edge_index, edge_weight, W0, W1, W2, b, gamma, beta):
    row = edge_index[0]
    col = edge_index[1]
    w = edge_weight

    degp = _deg_call(row, col, w)
    dis = _dis_call(degp).reshape(NP)

    xflat = jnp.zeros((NSC * NP, H), jnp.float32)
    xflat = lax.dynamic_update_slice(xflat, x[:, :H], (0, 0))
    xflat = lax.dynamic_update_slice(xflat, x[:, H:], (NP, 0))

    pad = EP - E
    rowp = jnp.concatenate([row, jnp.zeros((pad,), jnp.int32)])
    colp = jnp.concatenate([col, jnp.zeros((pad,), jnp.int32)])
    wp = jnp.concatenate([w, jnp.zeros((pad,), jnp.float32)])
    col2d = colp.reshape(EP // SUB, SUB)

    tx1_flat = _lmv_call(xflat, rowp, col2d, wp, dis)
    tx2_flat = _lmv_call(tx1_flat, rowp, col2d, wp, dis)

    t1 = tx1_flat.reshape(NSC, NP, H)[:, :N]
    t2 = tx2_flat.reshape(NSC, NP, H)[:, :N]

    out = _dense_call(x, t1, t2, W0, W1, W2,
                      b.reshape(1, D), gamma.reshape(1, D),
                      beta.reshape(1, D))
    return out


# speedup vs baseline: 1.0688x; 1.0003x over previous
import jax
import jax.numpy as jnp
from jax import lax
from jax.experimental import pallas as pl
from jax.experimental.pallas import tpu as pltpu
from jax.experimental.pallas import tpu_sc as plsc

N = 10000
E = 320000
D = 128
H = 64
NP = 10240
L = 16
NSC = 2
NSUBC = 16
ROWS_PER_TILE = NP // NSUBC

EP = 327680
EPT = EP // NSUBC
SUPER = 2048
NSUPER = EPT // SUPER
CH = 128
NCH = SUPER // CH
SUB = 128
SUBS_PER_SUPER = SUPER // SUB
NRING = 4

DCH = 400

_sc_mesh = plsc.VectorSubcoreMesh(core_axis_name="c", subcore_axis_name="s")


def _deg_body(row_hbm, col_hbm, w_hbm, out_hbm, rbuf, cbuf, wbuf, deg_v,
              dsem_a, dsem_b):
    c = lax.axis_index("c")
    s = lax.axis_index("s")
    wid = s * NSC + c
    dsems = (dsem_a, dsem_b)
    zeros = jnp.zeros((L,), jnp.float32)

    def zb(j, _):
        deg_v[pl.ds(j * L, L)] = zeros
        return 0

    lax.fori_loop(0, NP // L, zb, 0)

    epw = E // (NSC * NSUBC)
    nchunk = epw // DCH

    def descs(k, b):
        base = pl.multiple_of(wid * epw + k * DCH, 8)
        return (
            pltpu.make_async_copy(row_hbm.at[pl.ds(base, DCH)],
                                  rbuf.at[b], dsems[b]),
            pltpu.make_async_copy(col_hbm.at[pl.ds(base, DCH)],
                                  cbuf.at[b], dsems[b]),
            pltpu.make_async_copy(w_hbm.at[pl.ds(base, DCH)],
                                  wbuf.at[b], dsems[b]),
        )

    def fire(k, b):
        for dsc in descs(k, b):
            dsc.start()

    def wait(k, b):
        for dsc in descs(k, b):
            dsc.wait()

    def process(b):
        def ib(i, _):
            r = rbuf[b, pl.ds(i * L, L)]
            cv = cbuf[b, pl.ds(i * L, L)]
            wv = wbuf[b, pl.ds(i * L, L)]
            wsel = jnp.where(r == cv, 0.0, wv)
            plsc.addupdate_scatter(deg_v, [r], wsel)
            return 0

        lax.fori_loop(0, DCH // L, ib, 0)

    fire(0, 0)

    def pair(g, _):
        for b in (0, 1):
            k = 2 * g + b
            fire(k + 1, 1 - b)
            wait(k, b)
            process(b)
        return 0

    lax.fori_loop(0, (nchunk - 1) // 2, pair, 0)
    wait(nchunk - 1, 0)
    process(0)
    pltpu.sync_copy(deg_v, out_hbm.at[wid])


_sc_params = pltpu.CompilerParams(needs_layout_passes=False,
                                  use_tc_tiling_on_sc=False)

_deg_call = pl.kernel(
    _deg_body,
    out_type=jax.ShapeDtypeStruct((NSC * NSUBC, NP), jnp.float32),
    mesh=_sc_mesh,
    compiler_params=_sc_params,
    scratch_types=[
        pltpu.VMEM((2, DCH), jnp.int32),
        pltpu.VMEM((2, DCH), jnp.int32),
        pltpu.VMEM((2, DCH), jnp.float32),
        pltpu.VMEM((NP,), jnp.float32),
        pltpu.SemaphoreType.DMA,
        pltpu.SemaphoreType.DMA,
    ],
)


def _dis_body(degp_ref, dis_ref):
    deg = jnp.sum(degp_ref[...], axis=0, keepdims=True)
    dis_ref[...] = jnp.where(
        deg > 0, lax.rsqrt(jnp.maximum(deg, 1e-12)), 0.0)


def _dis_call(degp):
    return pl.pallas_call(
        _dis_body,
        out_shape=jax.ShapeDtypeStruct((1, NP), jnp.float32),
    )(degp)


def _lmv_body(src_hbm, row_hbm, col2d_hbm, w_hbm, dis_hbm, out_hbm,
              dis_v, ridx_s, w_s, cidx_s, gidx_s, wn_s,
              rows_a, rows_b, rows_c, rows_d,
              acc, lsem_a, lsem_b, gsem_a, gsem_b, gsem_c, gsem_d,
              ssem_a, ssem_b, ssem_c, ssem_d):
    c = lax.axis_index("c")
    s = lax.axis_index("s")
    lsems = (lsem_a, lsem_b)
    gsems = (gsem_a, gsem_b, gsem_c, gsem_d)
    ssems = (ssem_a, ssem_b, ssem_c, ssem_d)
    rows_bufs = (rows_a, rows_b, rows_c, rows_d)
    pltpu.sync_copy(dis_hbm, dis_v)
    zeros = jnp.zeros((L,), jnp.float32)

    def zb(i, _):
        for j in range(H // L):
            rows_a[i, pl.ds(j * L, L)] = zeros
        return 0

    lax.fori_loop(0, CH, zb, 0)
    off = 0
    while off < ROWS_PER_TILE:
        blk = min(CH, ROWS_PER_TILE - off)
        pltpu.sync_copy(rows_a.at[pl.ds(0, blk)],
                        acc.at[pl.ds(s * ROWS_PER_TILE + off, blk)])
        off += blk
    plsc.subcore_barrier()

    ebase = s * EPT
    coff = c * NP
    zero_i = jnp.zeros((L,), jnp.int32)

    def linear_descs(p, b):
        base = pl.multiple_of(ebase + p * SUPER, 8)
        rbase = pl.multiple_of((ebase + p * SUPER) // SUB, 8)
        return (
            pltpu.make_async_copy(row_hbm.at[pl.ds(base, SUPER)],
                                  ridx_s.at[b], lsems[b]),
            pltpu.make_async_copy(w_hbm.at[pl.ds(base, SUPER)],
                                  w_s.at[b], lsems[b]),
            pltpu.make_async_copy(col2d_hbm.at[pl.ds(rbase, SUBS_PER_SUPER)],
                                  cidx_s.at[b], lsems[b]),
        )

    def fire_linear(p, b):
        for dsc in linear_descs(p, b):
            dsc.start()

    def wait_linear(p, b):
        for dsc in linear_descs(p, b):
            dsc.wait()

    def gidx_compute(b):
        def gb(i):
            jb = i // (SUB // L)
            i2 = lax.rem(i, SUB // L)
            r = ridx_s[b, pl.ds(i * L, L)]
            gidx_s[b, jb, pl.ds(i2 * L, L)] = r + coff

        plsc.parallel_loop(0, SUPER // L, 1, unroll=4)(gb)

    def wn_compute(b):
        def wb(i):
            jb = i // (SUB // L)
            i2 = lax.rem(i, SUB // L)
            r = ridx_s[b, pl.ds(i * L, L)]
            cv = cidx_s[b, jb, pl.ds(i2 * L, L)]
            wv = w_s[b, pl.ds(i * L, L)]
            dr = plsc.load_gather(dis_v, [r])
            dc = plsc.load_gather(dis_v, [cv])
            wn = jnp.where(r == cv, 0.0, -dr * wv * dc)
            wn_s[b, pl.ds(i * L, L)] = wn

        plsc.parallel_loop(0, SUPER // L, 1, unroll=4)(wb)

    def gather_desc(b, k):
        return pltpu.make_async_copy(src_hbm.at[gidx_s.at[b, k]],
                                     rows_bufs[k % NRING], gsems[k % NRING])

    def scale_chunk(b, k, rbuf):
        koff = k * CH

        def sb(i):
            wni = plsc.load_gather(wn_s.at[b], [zero_i + (koff + i)])
            for jj in range(H // L):
                rbuf[i, pl.ds(jj * L, L)] = rbuf[i, pl.ds(jj * L, L)] * wni

        plsc.parallel_loop(0, CH, 1, unroll=8)(sb)

    def scatter_desc(b, k):
        return pltpu.async_copy(rows_bufs[k % NRING],
                                acc.at[cidx_s.at[b, k]],
                                ssems[k % NRING], add=True)

    def scatter_wait(b, k):
        pltpu.make_async_copy(rows_bufs[k % NRING],
                              acc.at[cidx_s.at[b, k]],
                              ssems[k % NRING]).wait()

    def do_super(p, b):
        @pl.when(p + 1 < NSUPER)
        def _():
            fire_linear(p + 1, 1 - b)

        wait_linear(p, b)
        gidx_compute(b)
        gather_desc(b, 0).start()
        gather_desc(b, 1).start()
        wn_compute(b)
        for k in range(NCH):
            if k + 2 < NCH:
                if k - 2 >= 0:
                    scatter_wait(b, k - 2)
                gather_desc(b, k + 2).start()
            gather_desc(b, k).wait()
            scale_chunk(b, k, rows_bufs[k % NRING])
            scatter_desc(b, k)
        for k in range(NCH - NRING, NCH):
            scatter_wait(b, k)

    fire_linear(0, 0)

    def gloop(g, _):
        do_super(2 * g, 0)
        do_super(2 * g + 1, 1)
        return 0

    lax.fori_loop(0, NSUPER // 2, gloop, 0)
    plsc.subcore_barrier()
    pltpu.sync_copy(acc.at[pl.ds(s * ROWS_PER_TILE, ROWS_PER_TILE)],
                    out_hbm.at[pl.ds(coff + s * ROWS_PER_TILE,
                                     ROWS_PER_TILE)])


_lmv_call = pl.kernel(
    _lmv_body,
    out_type=jax.ShapeDtypeStruct((NSC * NP, H), jnp.float32),
    mesh=_sc_mesh,
    compiler_params=_sc_params,
    scratch_types=[
        pltpu.VMEM((NP,), jnp.float32),
        pltpu.VMEM((2, SUPER), jnp.int32),
        pltpu.VMEM((2, SUPER), jnp.float32),
        pltpu.VMEM((2, SUBS_PER_SUPER, SUB), jnp.int32),
        pltpu.VMEM((2, SUBS_PER_SUPER, SUB), jnp.int32),
        pltpu.VMEM((2, SUPER), jnp.float32),
        pltpu.VMEM((CH, H), jnp.float32),
        pltpu.VMEM((CH, H), jnp.float32),
        pltpu.VMEM((CH, H), jnp.float32),
        pltpu.VMEM((CH, H), jnp.float32),
        pltpu.VMEM_SHARED((NP, H), jnp.float32),
    ] + [pltpu.SemaphoreType.DMA] * 10,
)


RB = 1000


def _dense_body(x_ref, t1_ref, t2_ref, w0_ref, w1_ref, w2_ref,
                b_ref, g_ref, be_ref, o_ref):
    xb = x_ref[...]
    t1 = jnp.concatenate([t1_ref[0], t1_ref[1]], axis=1)
    t2s = jnp.concatenate([t2_ref[0], t2_ref[1]], axis=1)
    t2 = 2.0 * t2s - xb
    acc = (jnp.dot(xb, w0_ref[...], preferred_element_type=jnp.float32)
           + jnp.dot(t1, w1_ref[...], preferred_element_type=jnp.float32)
           + jnp.dot(t2, w2_ref[...], preferred_element_type=jnp.float32)
           + b_ref[...])
    acc = jnp.maximum(acc, 0.0)
    mu = jnp.mean(acc, axis=1, keepdims=True)
    var = jnp.mean((acc - mu) ** 2, axis=1, keepdims=True)
    o_ref[...] = ((acc - mu) * lax.rsqrt(var + 1e-5) * g_ref[...]
                  + be_ref[...])


def _dense_call(x, t1, t2, W0, W1, W2, b2, g2, be2):
    grid = (N // RB,)
    return pl.pallas_call(
        _dense_body,
        grid=grid,
        in_specs=[
            pl.BlockSpec((RB, D), lambda i: (i, 0)),
            pl.BlockSpec((2, RB, H), lambda i: (0, i, 0)),
            pl.BlockSpec((2, RB, H), lambda i: (0, i, 0)),
            pl.BlockSpec((D, D), lambda i: (0, 0)),
            pl.BlockSpec((D, D), lambda i: (0, 0)),
            pl.BlockSpec((D, D), lambda i: (0, 0)),
            pl.BlockSpec((1, D), lambda i: (0, 0)),
            pl.BlockSpec((1, D), lambda i: (0, 0)),
            pl.BlockSpec((1, D), lambda i: (0, 0)),
        ],
        out_specs=pl.BlockSpec((RB, D), lambda i: (i, 0)),
        out_shape=jax.ShapeDtypeStruct((N, D), jnp.float32),
    )(x, t1, t2, W0, W1, W2, b2, g2, be2)


def kernel(x, edge_index, edge_weight, W0, W1, W2, b, gamma, beta):
    row = edge_index[0]
    col = edge_index[1]
    w = edge_weight

    degp = _deg_call(row, col, w)
    dis = _dis_call(degp).reshape(NP)

    xflat = jnp.zeros((NSC * NP, H), jnp.float32)
    xflat = lax.dynamic_update_slice(xflat, x[:, :H], (0, 0))
    xflat = lax.dynamic_update_slice(xflat, x[:, H:], (NP, 0))

    pad = EP - E
    rowp = jnp.concatenate([row, jnp.zeros((pad,), jnp.int32)])
    colp = jnp.concatenate([col, jnp.zeros((pad,), jnp.int32)])
    wp = jnp.concatenate([w, jnp.zeros((pad,), jnp.float32)])
    col2d = colp.reshape(EP // SUB, SUB)

    tx1_flat = _lmv_call(xflat, rowp, col2d, wp, dis)
    tx2_flat = _lmv_call(tx1_flat, rowp, col2d, wp, dis)

    t1 = tx1_flat.reshape(NSC, NP, H)[:, :N]
    t2 = tx2_flat.reshape(NSC, NP, H)[:, :N]

    out = _dense_call(x, t1, t2, W0, W1, W2,
                      b.reshape(1, D), gamma.reshape(1, D),
                      beta.reshape(1, D))
    return out
